# Initial kernel scaffold; baseline (speedup 1.0000x reference)
import jax
import jax.numpy as jnp
from jax.experimental import pallas as pl


def kernel(source, c0_directions, c1_weights, c1_bias, c1_directions, fc_w, fc_b, d1_weights, d1_bias, d1_directions, d2_weights, d2_bias, d2_directions, target_feature):
    raise NotImplementedError("write your pallas kernel here")


# speedup vs baseline: 18.8277x; 18.8277x over previous
import functools

import jax
import jax.numpy as jnp
from jax import lax
from jax.experimental import pallas as pl
from jax.experimental.pallas import tpu as pltpu
from jax.experimental.pallas import tpu_sc as plsc

BSZ = 8
NV = 2048
KNB = 20
RROWS = 256

def _knn_body(vr_ref, vw_ref, out_ref):
    vr = vr_ref[0]
    vw = vw_ref[0]
    inner = lax.dot_general(vr, vw, (((1,), (1,)), ((), ())),
                            preferred_element_type=jnp.float32)
    qr = jnp.sum(vr * vr, axis=1, keepdims=True)
    qc = jnp.sum(vw * vw, axis=1, keepdims=True).T
    acc = -2.0 * inner + qc + qr
    bits = lax.bitcast_convert_type(acc, jnp.int32)
    bits = jnp.bitwise_xor(
        bits,
        jnp.bitwise_and(lax.shift_right_arithmetic(bits, 31),
                        jnp.int32(0x7FFFFFFF)))
    w = lax.broadcasted_iota(jnp.int32, (RROWS, NV), 1)
    key = jnp.bitwise_or(jnp.bitwise_and(bits, jnp.int32(-2048)), w)
    lane32 = lax.broadcasted_iota(jnp.int32, (RROWS, 32), 1)

    def body(i, carry):
        last, accidx = carry
        cand = jnp.where(key > last, key, jnp.int32(0x7FFFFFFF))
        kmin = jnp.min(cand, axis=1, keepdims=True)
        idx = jnp.bitwise_and(kmin, jnp.int32(2047))
        accidx = jnp.where(lane32 == (i - 1), idx, accidx)
        return kmin, accidx

    last0 = jnp.full((RROWS, 1), -2 ** 31, jnp.int32)
    acc0 = jnp.zeros((RROWS, 32), jnp.int32)
    _, accidx = lax.fori_loop(0, KNB + 1, body, (last0, acc0))
    out_ref[0] = accidx[:, :KNB]


def _knn(verts):
    return pl.pallas_call(
        _knn_body,
        grid=(BSZ, NV // RROWS),
        in_specs=[
            pl.BlockSpec((1, RROWS, 3), lambda b, r: (b, r, 0)),
            pl.BlockSpec((1, NV, 3), lambda b, r: (b, 0, 0)),
        ],
        out_specs=pl.BlockSpec((1, RROWS, KNB), lambda b, r: (b, r, 0)),
        out_shape=jax.ShapeDtypeStruct((BSZ, NV, KNB), jnp.int32),
    )(verts, verts)


@functools.lru_cache(maxsize=None)
def _make_gather(C):
    TOT = BSZ * NV * KNB
    NW = 32
    CH = 128
    n_ch = TOT // (NW * CH)
    GRP = 16
    mesh = plsc.VectorSubcoreMesh(core_axis_name="c", subcore_axis_name="s")

    @functools.partial(
        pl.kernel, mesh=mesh,
        compiler_params=pltpu.CompilerParams(use_tc_tiling_on_sc=False),
        out_type=jax.ShapeDtypeStruct((TOT, C), jnp.float32),
        scratch_types=[
            pltpu.VMEM((n_ch, CH), jnp.int32),
            pltpu.VMEM((GRP, CH, C), jnp.float32),
            pltpu.SemaphoreType.DMA,
        ],
    )
    def gather_k(table_hbm, idx_hbm, out_hbm, idx_v, bufs, sem):
        wid = lax.axis_index("s") * 2 + lax.axis_index("c")
        pltpu.sync_copy(idx_hbm.at[pl.ds(wid * n_ch, n_ch)], idx_v)
        base = wid * n_ch * CH
        for g in range(n_ch // GRP):
            cps = [
                pltpu.async_copy(
                    table_hbm.at[idx_v.at[g * GRP + j]], bufs.at[j], sem)
                for j in range(GRP)
            ]
            for j in range(GRP):
                cps[j].wait()
                pltpu.sync_copy(
                    bufs.at[j],
                    out_hbm.at[pl.ds(base + (g * GRP + j) * CH, CH)])

    return gather_k


def _gather(table, idx2d, C):
    return _make_gather(C)(table, idx2d)


def _stage_a_body(g_ref, v_ref, sd_ref, w1_ref, b1_ref,
                  th1_ref, thd1_ref, thd2_ref, c1_ref, s1_ref):
    sd = sd_ref[...]
    cn = jnp.sqrt(jnp.sum(sd * sd, axis=0, keepdims=True))
    sdn = sd / jnp.maximum(cn, 1e-12)
    center = v_ref[0]
    f1 = None
    for n in range(KNB):
        nd = g_ref[0][:, 16 * n:16 * (n + 1)] - center
        s = jnp.sum(nd * nd, axis=1, keepdims=True)
        ndn = nd / jnp.maximum(jnp.sqrt(s), 1e-12)
        th = jax.nn.relu(
            lax.dot_general(ndn, sdn, (((1,), (0,)), ((), ())),
                            preferred_element_type=jnp.float32))
        th1_ref[0, :, 32 * n:32 * (n + 1)] = th[:, 32:64]
        thd1_ref[0, :, 16 * n:16 * (n + 1)] = th[:, 64:80]
        thd2_ref[0, :, 3 * n:3 * (n + 1)] = th[:, 80:83]
        f1 = th[:, :32] if n == 0 else jnp.maximum(f1, th[:, :32])
    fo = lax.dot_general(f1, w1_ref[...], (((1,), (0,)), ((), ())),
                         preferred_element_type=jnp.float32) + b1_ref[...]
    c1_ref[0] = fo[:, :32]
    s1_ref[0] = fo[:, 32:]


def _stage_a(gv, verts_p, sdp, w1, b1):
    return pl.pallas_call(
        _stage_a_body,
        grid=(BSZ,),
        in_specs=[
            pl.BlockSpec((1, NV, 16 * KNB), lambda b: (b, 0, 0)),
            pl.BlockSpec((1, NV, 16), lambda b: (b, 0, 0)),
            pl.BlockSpec((16, 83), lambda b: (0, 0)),
            pl.BlockSpec((32, 64), lambda b: (0, 0)),
            pl.BlockSpec((1, 64), lambda b: (0, 0)),
        ],
        out_specs=[
            pl.BlockSpec((1, NV, 32 * KNB), lambda b: (b, 0, 0)),
            pl.BlockSpec((1, NV, 16 * KNB), lambda b: (b, 0, 0)),
            pl.BlockSpec((1, NV, 3 * KNB), lambda b: (b, 0, 0)),
            pl.BlockSpec((1, NV, 32), lambda b: (b, 0, 0)),
            pl.BlockSpec((1, NV, 32), lambda b: (b, 0, 0)),
        ],
        out_shape=[
            jax.ShapeDtypeStruct((BSZ, NV, 32 * KNB), jnp.float32),
            jax.ShapeDtypeStruct((BSZ, NV, 16 * KNB), jnp.float32),
            jax.ShapeDtypeStruct((BSZ, NV, 3 * KNB), jnp.float32),
            jax.ShapeDtypeStruct((BSZ, NV, 32), jnp.float32),
            jax.ShapeDtypeStruct((BSZ, NV, 32), jnp.float32),
        ],
    )(gv, verts_p, sdp, w1, b1)


def _stage_c_body(th_ref, g_ref, c_ref, tf_ref, fw_ref, fb_ref, w_ref, b_ref,
                  cd_ref, sd_ref):
    act = None
    for n in range(KNB):
        p = th_ref[0][:, 32 * n:32 * (n + 1)] * g_ref[0][:, 32 * n:32 * (n + 1)]
        act = p if n == 0 else jnp.maximum(act, p)
    f2 = jax.nn.relu(c_ref[0] + act)
    mean = jnp.mean(f2, axis=0, keepdims=True)
    xc = f2 - mean
    std = jnp.sqrt(jnp.sum(xc * xc, axis=0, keepdims=True) / (NV - 1)) + 1e-8
    xn = xc / std
    h = lax.dot_general(tf_ref[0], fw_ref[...], (((1,), (0,)), ((), ())),
                        preferred_element_type=jnp.float32) + fb_ref[...]
    t = (1.0 + h[:, :32]) * xn + h[:, 32:]
    fo = lax.dot_general(t, w_ref[...], (((1,), (0,)), ((), ())),
                         preferred_element_type=jnp.float32) + b_ref[...]
    cd_ref[0] = fo[:, :16]
    sd_ref[0] = fo[:, 16:]


def _stage_c(th1, g1, c1c, tf, fw_t, fb, d1w, d1b):
    return pl.pallas_call(
        _stage_c_body,
        grid=(BSZ,),
        in_specs=[
            pl.BlockSpec((1, NV, 32 * KNB), lambda b: (b, 0, 0)),
            pl.BlockSpec((1, NV, 32 * KNB), lambda b: (b, 0, 0)),
            pl.BlockSpec((1, NV, 32), lambda b: (b, 0, 0)),
            pl.BlockSpec((1, NV, 10), lambda b: (b, 0, 0)),
            pl.BlockSpec((10, 64), lambda b: (0, 0)),
            pl.BlockSpec((1, 64), lambda b: (0, 0)),
            pl.BlockSpec((32, 32), lambda b: (0, 0)),
            pl.BlockSpec((1, 32), lambda b: (0, 0)),
        ],
        out_specs=[
            pl.BlockSpec((1, NV, 16), lambda b: (b, 0, 0)),
            pl.BlockSpec((1, NV, 16), lambda b: (b, 0, 0)),
        ],
        out_shape=[
            jax.ShapeDtypeStruct((BSZ, NV, 16), jnp.float32),
            jax.ShapeDtypeStruct((BSZ, NV, 16), jnp.float32),
        ],
    )(th1, g1, c1c, tf, fw_t, fb, d1w, d1b)


def _stage_d_body(th_ref, g_ref, c_ref, w_ref, b_ref, cd_ref, sd_ref):
    act = None
    for n in range(KNB):
        p = th_ref[0][:, 16 * n:16 * (n + 1)] * g_ref[0][:, 16 * n:16 * (n + 1)]
        act = p if n == 0 else jnp.maximum(act, p)
    c1o = jax.nn.relu(c_ref[0] + act)
    fo = lax.dot_general(c1o, w_ref[...], (((1,), (0,)), ((), ())),
                         preferred_element_type=jnp.float32) + b_ref[...]
    cd_ref[0] = fo[:, :3]
    sd_ref[0] = fo[:, 3:19]


def _stage_d(thd1, g2, cd1, d2w_p, d2b_p):
    return pl.pallas_call(
        _stage_d_body,
        grid=(BSZ,),
        in_specs=[
            pl.BlockSpec((1, NV, 16 * KNB), lambda b: (b, 0, 0)),
            pl.BlockSpec((1, NV, 16 * KNB), lambda b: (b, 0, 0)),
            pl.BlockSpec((1, NV, 16), lambda b: (b, 0, 0)),
            pl.BlockSpec((16, 19), lambda b: (0, 0)),
            pl.BlockSpec((1, 19), lambda b: (0, 0)),
        ],
        out_specs=[
            pl.BlockSpec((1, NV, 3), lambda b: (b, 0, 0)),
            pl.BlockSpec((1, NV, 16), lambda b: (b, 0, 0)),
        ],
        out_shape=[
            jax.ShapeDtypeStruct((BSZ, NV, 3), jnp.float32),
            jax.ShapeDtypeStruct((BSZ, NV, 16), jnp.float32),
        ],
    )(thd1, g2, cd1, d2w_p, d2b_p)


def _stage_e_body(th_ref, g_ref, c_ref, out_ref):
    act = None
    for n in range(KNB):
        p = th_ref[0][:, 3 * n:3 * (n + 1)] * g_ref[0][:, 16 * n:16 * n + 3]
        act = p if n == 0 else jnp.maximum(act, p)
    out_ref[0] = jax.nn.sigmoid(c_ref[0] + act)


def _stage_e(thd2, g3, cd2):
    return pl.pallas_call(
        _stage_e_body,
        grid=(BSZ,),
        in_specs=[
            pl.BlockSpec((1, NV, 3 * KNB), lambda b: (b, 0, 0)),
            pl.BlockSpec((1, NV, 16 * KNB), lambda b: (b, 0, 0)),
            pl.BlockSpec((1, NV, 3), lambda b: (b, 0, 0)),
        ],
        out_specs=pl.BlockSpec((1, NV, 3), lambda b: (b, 0, 0)),
        out_shape=jax.ShapeDtypeStruct((BSZ, NV, 3), jnp.float32),
    )(thd2, g3, cd2)


def kernel(source, c0_directions, c1_weights, c1_bias, c1_directions, fc_w,
           fc_b, d1_weights, d1_bias, d1_directions, d2_weights, d2_bias,
           d2_directions, target_feature):
    idx = _knn(source)
    bias = (jnp.arange(BSZ, dtype=jnp.int32) * NV)[:, None, None]
    idx2d = jnp.reshape(idx + bias, (-1, 128))

    verts_p = jnp.pad(source, ((0, 0), (0, 0), (0, 13)))
    gv = _gather(jnp.reshape(verts_p, (BSZ * NV, 16)), idx2d, 16)
    gv = jnp.reshape(gv, (BSZ, NV, 16 * KNB))

    sd_all = jnp.concatenate(
        [c0_directions, c1_directions, d1_directions, d2_directions], axis=1)
    sd_p = jnp.pad(sd_all, ((0, 13), (0, 0)))
    th1, thd1, thd2, c1c, s1 = _stage_a(
        gv, verts_p, sd_p, c1_weights, jnp.reshape(c1_bias, (1, -1)))

    g1 = _gather(jnp.reshape(s1, (BSZ * NV, 32)), idx2d, 32)
    g1 = jnp.reshape(g1, (BSZ, NV, 32 * KNB))
    cd1, sup_d1 = _stage_c(
        th1, g1, c1c, target_feature, jnp.transpose(fc_w),
        jnp.reshape(fc_b, (1, -1)), d1_weights, jnp.reshape(d1_bias, (1, -1)))

    g2 = _gather(jnp.reshape(sup_d1, (BSZ * NV, 16)), idx2d, 16)
    g2 = jnp.reshape(g2, (BSZ, NV, 16 * KNB))
    d2w_p = jnp.pad(d2_weights, ((0, 0), (0, 13)))
    d2b_p = jnp.reshape(jnp.pad(d2_bias, (0, 13)), (1, -1))
    cd2, sup_d2 = _stage_d(thd1, g2, cd1, d2w_p, d2b_p)

    g3 = _gather(jnp.reshape(sup_d2, (BSZ * NV, 16)), idx2d, 16)
    g3 = jnp.reshape(g3, (BSZ, NV, 16 * KNB))
    return _stage_e(thd2, g3, cd2)
